# baseline (speedup 1.0000x reference)
import functools

import jax
import jax.numpy as jnp
from jax import lax
from jax.experimental import pallas as pl
from jax.experimental.pallas import tpu as pltpu
from jax.experimental.pallas import tpu_sc as plsc

EMBED_DIM = 64
_LANES = 16

_mesh = plsc.VectorSubcoreMesh(
    core_axis_name="c", subcore_axis_name="s", num_cores=1
)


@functools.partial(
    pl.kernel,
    mesh=_mesh,
    out_type=jax.ShapeDtypeStruct((1, EMBED_DIM), jnp.float32),
    scratch_types=[
        pltpu.VMEM((_LANES,), jnp.int32),
    ],
)
def _sc_lookup(ids_hbm, table_hbm, out_hbm, idx_v):
    s = lax.axis_index("s")

    @pl.when(s == 0)
    def _():
        pltpu.sync_copy(ids_hbm.at[0, pl.ds(0, _LANES)], idx_v)
        idx0 = idx_v[...][0]
        pltpu.sync_copy(table_hbm.at[pl.ds(idx0, 1), :], out_hbm)


def kernel(ids, table):
    return _sc_lookup(ids.astype(jnp.int32), table)[0]
